# baseline (speedup 1.0000x reference)
import functools

import jax
import jax.numpy as jnp
from jax import lax
from jax.experimental import pallas as pl
from jax.experimental.pallas import tpu as pltpu
from jax.experimental.pallas import tpu_sc as plsc

N = 1024
W = 5
D = 64
K = 20
H = 128
B = 256

_NB = 16
_NW = 32
_S = 1
_NCN = N // _S
_ROWS_PER_W = (_NCN * K) // _NW
_CHUNK = 64
_NCHUNK = _ROWS_PER_W // _CHUNK
_HB = B // 2


def _topk_embproj_body(emb_ref, w1b_ref, b1_ref, x_ref, topk_ref, proj_ref,
                       xt_ref):
    mask_hi = jnp.int32(-65536)
    for t in range(W):
        xt_t = jnp.transpose(x_ref[:, t, :])
        rb = lax.bitcast_convert_type(
            xt_t.astype(jnp.bfloat16).astype(jnp.float32), jnp.int32)
        lo = lax.shift_right_logical(rb[:, :_HB], 16)
        hi = lax.bitwise_and(rb[:, _HB:], mask_hi)
        xt_ref[:, t * _HB:(t + 1) * _HB] = lax.bitwise_or(lo, hi)
    emb = emb_ref[...]
    nrm = jnp.sqrt(jnp.sum(emb * emb, axis=1, keepdims=True))
    norm = emb / (nrm + 1e-12)
    sim = lax.dot_general(norm, norm, (((1,), (1,)), ((), ())),
                          preferred_element_type=jnp.float32)
    row = lax.broadcasted_iota(jnp.int32, (N, N), 0)
    col = lax.broadcasted_iota(jnp.int32, (N, N), 1)
    sim = sim - jnp.where(row == col, jnp.float32(1e9), jnp.float32(0.0))
    cols = []
    for _ in range(K):
        m = jnp.max(sim, axis=1, keepdims=True)
        cand = jnp.where(sim == m, col, jnp.int32(N))
        idxk = jnp.min(cand, axis=1, keepdims=True)
        cols.append(idxk)
        sim = jnp.where(col == idxk, jnp.float32(-jnp.inf), sim)
    topk_ref[...] = jnp.concatenate(cols, axis=1)
    proj_ref[...] = lax.dot_general(
        w1b_ref[...], emb, (((0,), (1,)), ((), ())),
        preferred_element_type=jnp.float32) + b1_ref[...][:, None]


def _topk_embproj(emb, W1b, b1, x):
    return pl.pallas_call(
        _topk_embproj_body,
        out_shape=(jax.ShapeDtypeStruct((N, K), jnp.int32),
                   jax.ShapeDtypeStruct((H, N), jnp.float32),
                   jax.ShapeDtypeStruct((N, W * _HB), jnp.int32)),
    )(emb, W1b, b1, x)


def _sc_gather(xt_packed, idx_flat):
    mesh = plsc.VectorSubcoreMesh(core_axis_name="c", subcore_axis_name="s")

    @functools.partial(
        pl.kernel,
        mesh=mesh,
        out_type=jax.ShapeDtypeStruct((_NCN * K, W * _HB), jnp.int32),
        scratch_types=[
            pltpu.VMEM((_NCHUNK, _CHUNK), jnp.int32),
            pltpu.VMEM((_CHUNK, W * _HB), jnp.int32),
            pltpu.VMEM((_CHUNK, W * _HB), jnp.int32),
            pltpu.VMEM((_CHUNK, W * _HB), jnp.int32),
            pltpu.SemaphoreType.DMA,
            pltpu.SemaphoreType.DMA,
            pltpu.SemaphoreType.DMA,
            pltpu.SemaphoreType.DMA,
            pltpu.SemaphoreType.DMA,
            pltpu.SemaphoreType.DMA,
        ],
    )
    def k(table_hbm, idx_hbm, out_hbm, idx_v,
          rows0, rows1, rows2, gs0, gs1, gs2, ws0, ws1, ws2):
        wid = lax.axis_index("s") * 2 + lax.axis_index("c")
        base = wid * _ROWS_PER_W
        for c in range(_NCHUNK):
            pltpu.sync_copy(idx_hbm.at[pl.ds(base + c * _CHUNK, _CHUNK)],
                            idx_v.at[c])
        bufs = (rows0, rows1, rows2)
        gsems = (gs0, gs1, gs2)
        wsems = (ws0, ws1, ws2)
        gcp = [None] * _NCHUNK
        wcp = [None] * _NCHUNK

        def start_gather(c):
            gcp[c] = pltpu.async_copy(table_hbm.at[idx_v.at[c]],
                                      bufs[c % 3], gsems[c % 3])

        start_gather(0)
        if _NCHUNK > 1:
            start_gather(1)
        for c in range(_NCHUNK):
            gcp[c].wait()
            wcp[c] = pltpu.async_copy(
                bufs[c % 3], out_hbm.at[pl.ds(base + c * _CHUNK, _CHUNK)],
                wsems[c % 3])
            if c + 2 < _NCHUNK:
                if wcp[c + 2 - 3] is not None:
                    wcp[c + 2 - 3].wait()
                start_gather(c + 2)
        for c in range(max(0, _NCHUNK - 3), _NCHUNK):
            if wcp[c] is not None:
                wcp[c].wait()

    return k(xt_packed, idx_flat)


def _mlp_body(g_ref, proj_ref, w1t_ref, w2t_ref, b2_ref, w3t_ref, b3_ref, out_ref):
    w1t = w1t_ref[...]
    w2t = w2t_ref[...]
    b2c = b2_ref[...][:, None]
    w3t = w3t_ref[...]
    b3 = b3_ref[0]
    mask_hi = jnp.int32(-65536)
    pieces = []
    for n in range(_NB):
        gi = g_ref[n]
        lo = lax.bitcast_convert_type(lax.shift_left(gi, 16), jnp.float32)
        hi = lax.bitcast_convert_type(lax.bitwise_and(gi, mask_hi), jnp.float32)
        pieces.append(lo)
        pieces.append(hi)
    g = jnp.concatenate(pieces, axis=1).astype(jnp.bfloat16)
    h1 = lax.dot_general(w1t, g, (((1,), (0,)), ((), ())),
                         preferred_element_type=jnp.float32)
    projb = jnp.concatenate(
        [jnp.broadcast_to(proj_ref[0, :, n][:, None], (H, B))
         for n in range(_NB)], axis=1)
    h1 = jnp.maximum(h1 + projb, 0.0)
    h2 = jnp.maximum(
        lax.dot_general(w2t, h1.astype(jnp.bfloat16), (((1,), (0,)), ((), ())),
                        preferred_element_type=jnp.float32) + b2c,
        0.0)
    o = lax.dot_general(w3t, h2, (((1,), (0,)), ((), ())),
                        preferred_element_type=jnp.float32)
    out_ref[0] = o + b3


def _mlp(gathered, projT, W1t, W2t, b2, W3t, b3):
    return pl.pallas_call(
        _mlp_body,
        grid=(_NCN // _NB,),
        in_specs=[
            pl.BlockSpec((_NB, K * W, _HB), lambda i: (i, 0, 0)),
            pl.BlockSpec((1, H, _NB), lambda i: (i, 0, 0)),
            pl.BlockSpec((H, K * W), lambda i: (0, 0)),
            pl.BlockSpec((H // 2, H), lambda i: (0, 0)),
            pl.BlockSpec((H // 2,), lambda i: (0,)),
            pl.BlockSpec((1, H // 2), lambda i: (0, 0)),
            pl.BlockSpec(memory_space=pltpu.SMEM),
        ],
        out_specs=pl.BlockSpec((1, 1, _NB * B), lambda i: (i, 0, 0)),
        out_shape=jax.ShapeDtypeStruct((_NCN // _NB, 1, _NB * B), jnp.float32),
    )(gathered, projT, W1t, W2t, b2, W3t, b3)


def kernel(x, emb, W1, b1, W2, b2, W3, b3):
    W1b = W1[K * W:]
    W1t = W1[:K * W].reshape(W, K, H).transpose(2, 1, 0).reshape(H, K * W)
    W1t = W1t.astype(jnp.bfloat16)
    topk, projT, xt_packed = _topk_embproj(emb, W1b, b1, x)
    projT3 = projT.reshape(H, N // _NB, _NB).transpose(1, 0, 2)
    topk_flat = topk.reshape(-1)
    W2tb = W2.T.astype(jnp.bfloat16)
    W3t = W3.T
    blocks_per_s = _NCN // _NB
    outs = []
    for s in range(_S):
        g_s = _sc_gather(xt_packed,
                         topk_flat[s * _NCN * K:(s + 1) * _NCN * K])
        g_s = g_s.reshape(_NCN, K * W, _HB)
        pT_s = projT3[s * blocks_per_s:(s + 1) * blocks_per_s]
        outs.append(_mlp(g_s, pT_s, W1t, W2tb, b2, W3t, b3))
    out2 = jnp.concatenate(outs, axis=0)
    return out2.reshape(N, B).T


# speedup vs baseline: 1.1463x; 1.1463x over previous
import functools

import jax
import jax.numpy as jnp
from jax import lax
from jax.experimental import pallas as pl
from jax.experimental.pallas import tpu as pltpu
from jax.experimental.pallas import tpu_sc as plsc

N = 1024
W = 5
D = 64
K = 20
H = 128
B = 256

_NB = 32
_NW = 32
_S = 1
_NCN = N // _S
_ROWS_PER_W = (_NCN * K) // _NW
_CHUNK = 64
_NCHUNK = _ROWS_PER_W // _CHUNK
_HB = B // 2


def _topk_embproj_body(emb_ref, w1b_ref, b1_ref, topk_ref, proj_ref):
    emb = emb_ref[...]
    nrm = jnp.sqrt(jnp.sum(emb * emb, axis=1, keepdims=True))
    norm = emb / (nrm + 1e-12)
    sim = lax.dot_general(norm, norm, (((1,), (1,)), ((), ())),
                          preferred_element_type=jnp.float32)
    row = lax.broadcasted_iota(jnp.int32, (N, N), 0)
    col = lax.broadcasted_iota(jnp.int32, (N, N), 1)
    sim = sim - jnp.where(row == col, jnp.float32(1e9), jnp.float32(0.0))
    cols = []
    for _ in range(K):
        m = jnp.max(sim, axis=1, keepdims=True)
        cand = jnp.where(sim == m, col, jnp.int32(N))
        idxk = jnp.min(cand, axis=1, keepdims=True)
        cols.append(idxk)
        sim = jnp.where(col == idxk, jnp.float32(-jnp.inf), sim)
    topk_ref[...] = jnp.concatenate(cols, axis=1)
    proj_ref[...] = lax.dot_general(
        w1b_ref[...], emb, (((0,), (1,)), ((), ())),
        preferred_element_type=jnp.float32) + b1_ref[...][:, None]


def _topk_embproj(emb, W1b, b1):
    return pl.pallas_call(
        _topk_embproj_body,
        out_shape=(jax.ShapeDtypeStruct((N, K), jnp.int32),
                   jax.ShapeDtypeStruct((H, N), jnp.float32)),
    )(emb, W1b, b1)


def _sc_gather(xt_packed, idx_flat):
    mesh = plsc.VectorSubcoreMesh(core_axis_name="c", subcore_axis_name="s")

    @functools.partial(
        pl.kernel,
        mesh=mesh,
        out_type=jax.ShapeDtypeStruct((_NCN * K, W * _HB), jnp.int32),
        scratch_types=[
            pltpu.VMEM((_NCHUNK, _CHUNK), jnp.int32),
            pltpu.VMEM((_CHUNK, W * _HB), jnp.int32),
            pltpu.VMEM((_CHUNK, W * _HB), jnp.int32),
            pltpu.VMEM((_CHUNK, W * _HB), jnp.int32),
            pltpu.SemaphoreType.DMA,
            pltpu.SemaphoreType.DMA,
            pltpu.SemaphoreType.DMA,
            pltpu.SemaphoreType.DMA,
            pltpu.SemaphoreType.DMA,
            pltpu.SemaphoreType.DMA,
        ],
    )
    def k(table_hbm, idx_hbm, out_hbm, idx_v,
          rows0, rows1, rows2, gs0, gs1, gs2, ws0, ws1, ws2):
        wid = lax.axis_index("s") * 2 + lax.axis_index("c")
        base = wid * _ROWS_PER_W
        for c in range(_NCHUNK):
            pltpu.sync_copy(idx_hbm.at[pl.ds(base + c * _CHUNK, _CHUNK)],
                            idx_v.at[c])
        bufs = (rows0, rows1, rows2)
        gsems = (gs0, gs1, gs2)
        wsems = (ws0, ws1, ws2)
        gcp = [None] * _NCHUNK
        wcp = [None] * _NCHUNK

        def start_gather(c):
            gcp[c] = pltpu.async_copy(table_hbm.at[idx_v.at[c]],
                                      bufs[c % 3], gsems[c % 3])

        start_gather(0)
        if _NCHUNK > 1:
            start_gather(1)
        for c in range(_NCHUNK):
            gcp[c].wait()
            wcp[c] = pltpu.async_copy(
                bufs[c % 3], out_hbm.at[pl.ds(base + c * _CHUNK, _CHUNK)],
                wsems[c % 3])
            if c + 2 < _NCHUNK:
                if wcp[c + 2 - 3] is not None:
                    wcp[c + 2 - 3].wait()
                start_gather(c + 2)
        for c in range(max(0, _NCHUNK - 3), _NCHUNK):
            if wcp[c] is not None:
                wcp[c].wait()

    return k(xt_packed, idx_flat)


def _mlp_body(g_ref, proj_ref, w1t_ref, w2t_ref, b2_ref, w3t_ref, b3_ref, out_ref):
    w1t = w1t_ref[...]
    w2t = w2t_ref[...]
    b2c = b2_ref[...][:, None]
    w3t = w3t_ref[...]
    b3 = b3_ref[0]
    mask_hi = jnp.int32(-65536)
    pieces = []
    for n in range(_NB):
        gi = g_ref[n]
        lo = lax.bitcast_convert_type(lax.shift_left(gi, 16), jnp.float32)
        hi = lax.bitcast_convert_type(lax.bitwise_and(gi, mask_hi), jnp.float32)
        pieces.append(lo)
        pieces.append(hi)
    g = jnp.concatenate(pieces, axis=1).astype(jnp.bfloat16)
    h1 = lax.dot_general(w1t, g, (((1,), (0,)), ((), ())),
                         preferred_element_type=jnp.float32)
    projb = jnp.concatenate(
        [jnp.broadcast_to(proj_ref[0, :, n][:, None], (H, B))
         for n in range(_NB)], axis=1)
    h1 = jnp.maximum(h1 + projb, 0.0)
    h2 = jnp.maximum(
        lax.dot_general(w2t, h1.astype(jnp.bfloat16), (((1,), (0,)), ((), ())),
                        preferred_element_type=jnp.float32) + b2c,
        0.0)
    o = lax.dot_general(w3t, h2, (((1,), (0,)), ((), ())),
                        preferred_element_type=jnp.float32)
    out_ref[0] = o + b3


def _mlp(gathered, projT, W1t, W2t, b2, W3t, b3):
    return pl.pallas_call(
        _mlp_body,
        grid=(_NCN // _NB,),
        in_specs=[
            pl.BlockSpec((_NB, K * W, _HB), lambda i: (i, 0, 0)),
            pl.BlockSpec((1, H, _NB), lambda i: (i, 0, 0)),
            pl.BlockSpec((H, K * W), lambda i: (0, 0)),
            pl.BlockSpec((H // 2, H), lambda i: (0, 0)),
            pl.BlockSpec((H // 2,), lambda i: (0,)),
            pl.BlockSpec((1, H // 2), lambda i: (0, 0)),
            pl.BlockSpec(memory_space=pltpu.SMEM),
        ],
        out_specs=pl.BlockSpec((1, 1, _NB * B), lambda i: (i, 0, 0)),
        out_shape=jax.ShapeDtypeStruct((_NCN // _NB, 1, _NB * B), jnp.float32),
    )(gathered, projT, W1t, W2t, b2, W3t, b3)


def kernel(x, emb, W1, b1, W2, b2, W3, b3):
    W1b = W1[K * W:]
    W1t = W1[:K * W].reshape(W, K, H).transpose(2, 1, 0).reshape(H, K * W)
    W1t = W1t.astype(jnp.bfloat16)
    xt = jnp.transpose(x, (2, 1, 0)).astype(jnp.bfloat16)
    xt = xt.reshape(N, W, 2, _HB).transpose(0, 1, 3, 2)
    xt_packed = lax.bitcast_convert_type(xt, jnp.int32)
    xt_packed = xt_packed.reshape(N, W * _HB)

    topk, projT = _topk_embproj(emb, W1b, b1)
    projT3 = projT.reshape(H, N // _NB, _NB).transpose(1, 0, 2)
    topk_flat = topk.reshape(-1)
    W2tb = W2.T.astype(jnp.bfloat16)
    W3t = W3.T
    blocks_per_s = _NCN // _NB
    outs = []
    for s in range(_S):
        g_s = _sc_gather(xt_packed,
                         topk_flat[s * _NCN * K:(s + 1) * _NCN * K])
        g_s = g_s.reshape(_NCN, K * W, _HB)
        pT_s = projT3[s * blocks_per_s:(s + 1) * blocks_per_s]
        outs.append(_mlp(g_s, pT_s, W1t, W2tb, b2, W3t, b3))
    out2 = jnp.concatenate(outs, axis=0)
    return out2.reshape(N, B).T


# speedup vs baseline: 1.1813x; 1.0305x over previous
import functools

import jax
import jax.numpy as jnp
from jax import lax
from jax.experimental import pallas as pl
from jax.experimental.pallas import tpu as pltpu
from jax.experimental.pallas import tpu_sc as plsc

N = 1024
W = 5
D = 64
K = 20
H = 128
B = 256

_NB = 64
_NW = 32
_S = 1
_NCN = N // _S
_ROWS_PER_W = (_NCN * K) // _NW
_CHUNK = 64
_NCHUNK = _ROWS_PER_W // _CHUNK
_HB = B // 2


def _topk_embproj_body(emb_ref, w1b_ref, b1_ref, topk_ref, proj_ref):
    emb = emb_ref[...]
    nrm = jnp.sqrt(jnp.sum(emb * emb, axis=1, keepdims=True))
    norm = emb / (nrm + 1e-12)
    sim = lax.dot_general(norm, norm, (((1,), (1,)), ((), ())),
                          preferred_element_type=jnp.float32)
    row = lax.broadcasted_iota(jnp.int32, (N, N), 0)
    col = lax.broadcasted_iota(jnp.int32, (N, N), 1)
    sim = sim - jnp.where(row == col, jnp.float32(1e9), jnp.float32(0.0))
    cols = []
    for _ in range(K):
        m = jnp.max(sim, axis=1, keepdims=True)
        cand = jnp.where(sim == m, col, jnp.int32(N))
        idxk = jnp.min(cand, axis=1, keepdims=True)
        cols.append(idxk)
        sim = jnp.where(col == idxk, jnp.float32(-jnp.inf), sim)
    topk_ref[...] = jnp.concatenate(cols, axis=1)
    proj_ref[...] = lax.dot_general(
        w1b_ref[...], emb, (((0,), (1,)), ((), ())),
        preferred_element_type=jnp.float32) + b1_ref[...][:, None]


def _topk_embproj(emb, W1b, b1):
    return pl.pallas_call(
        _topk_embproj_body,
        out_shape=(jax.ShapeDtypeStruct((N, K), jnp.int32),
                   jax.ShapeDtypeStruct((H, N), jnp.float32)),
    )(emb, W1b, b1)


def _sc_gather(xt_packed, idx_flat):
    mesh = plsc.VectorSubcoreMesh(core_axis_name="c", subcore_axis_name="s")

    @functools.partial(
        pl.kernel,
        mesh=mesh,
        out_type=jax.ShapeDtypeStruct((_NCN * K, W * _HB), jnp.int32),
        scratch_types=[
            pltpu.VMEM((_NCHUNK, _CHUNK), jnp.int32),
            pltpu.VMEM((_CHUNK, W * _HB), jnp.int32),
            pltpu.VMEM((_CHUNK, W * _HB), jnp.int32),
            pltpu.VMEM((_CHUNK, W * _HB), jnp.int32),
            pltpu.SemaphoreType.DMA,
            pltpu.SemaphoreType.DMA,
            pltpu.SemaphoreType.DMA,
            pltpu.SemaphoreType.DMA,
            pltpu.SemaphoreType.DMA,
            pltpu.SemaphoreType.DMA,
        ],
    )
    def k(table_hbm, idx_hbm, out_hbm, idx_v,
          rows0, rows1, rows2, gs0, gs1, gs2, ws0, ws1, ws2):
        wid = lax.axis_index("s") * 2 + lax.axis_index("c")
        base = wid * _ROWS_PER_W
        for c in range(_NCHUNK):
            pltpu.sync_copy(idx_hbm.at[pl.ds(base + c * _CHUNK, _CHUNK)],
                            idx_v.at[c])
        bufs = (rows0, rows1, rows2)
        gsems = (gs0, gs1, gs2)
        wsems = (ws0, ws1, ws2)
        gcp = [None] * _NCHUNK
        wcp = [None] * _NCHUNK

        def start_gather(c):
            gcp[c] = pltpu.async_copy(table_hbm.at[idx_v.at[c]],
                                      bufs[c % 3], gsems[c % 3])

        start_gather(0)
        if _NCHUNK > 1:
            start_gather(1)
        for c in range(_NCHUNK):
            gcp[c].wait()
            wcp[c] = pltpu.async_copy(
                bufs[c % 3], out_hbm.at[pl.ds(base + c * _CHUNK, _CHUNK)],
                wsems[c % 3])
            if c + 2 < _NCHUNK:
                if wcp[c + 2 - 3] is not None:
                    wcp[c + 2 - 3].wait()
                start_gather(c + 2)
        for c in range(max(0, _NCHUNK - 3), _NCHUNK):
            if wcp[c] is not None:
                wcp[c].wait()

    return k(xt_packed, idx_flat)


def _mlp_body(g_ref, proj_ref, w1t_ref, w2t_ref, b2_ref, w3t_ref, b3_ref, out_ref):
    w1t = w1t_ref[...]
    w2t = w2t_ref[...]
    b2c = b2_ref[...][:, None]
    w3t = w3t_ref[...]
    b3 = b3_ref[0]
    mask_hi = jnp.int32(-65536)
    pieces = []
    for n in range(_NB):
        gi = g_ref[n]
        lo = lax.bitcast_convert_type(lax.shift_left(gi, 16), jnp.float32)
        hi = lax.bitcast_convert_type(lax.bitwise_and(gi, mask_hi), jnp.float32)
        pieces.append(lo)
        pieces.append(hi)
    g = jnp.concatenate(pieces, axis=1).astype(jnp.bfloat16)
    h1 = lax.dot_general(w1t, g, (((1,), (0,)), ((), ())),
                         preferred_element_type=jnp.float32)
    projb = jnp.concatenate(
        [jnp.broadcast_to(proj_ref[0, :, n][:, None], (H, B))
         for n in range(_NB)], axis=1)
    h1 = jnp.maximum(h1 + projb, 0.0)
    h2 = jnp.maximum(
        lax.dot_general(w2t, h1.astype(jnp.bfloat16), (((1,), (0,)), ((), ())),
                        preferred_element_type=jnp.float32) + b2c,
        0.0)
    o = lax.dot_general(w3t, h2, (((1,), (0,)), ((), ())),
                        preferred_element_type=jnp.float32)
    out_ref[0] = o + b3


def _mlp(gathered, projT, W1t, W2t, b2, W3t, b3):
    return pl.pallas_call(
        _mlp_body,
        grid=(_NCN // _NB,),
        in_specs=[
            pl.BlockSpec((_NB, K * W, _HB), lambda i: (i, 0, 0)),
            pl.BlockSpec((1, H, _NB), lambda i: (i, 0, 0)),
            pl.BlockSpec((H, K * W), lambda i: (0, 0)),
            pl.BlockSpec((H // 2, H), lambda i: (0, 0)),
            pl.BlockSpec((H // 2,), lambda i: (0,)),
            pl.BlockSpec((1, H // 2), lambda i: (0, 0)),
            pl.BlockSpec(memory_space=pltpu.SMEM),
        ],
        out_specs=pl.BlockSpec((1, 1, _NB * B), lambda i: (i, 0, 0)),
        out_shape=jax.ShapeDtypeStruct((_NCN // _NB, 1, _NB * B), jnp.float32),
    )(gathered, projT, W1t, W2t, b2, W3t, b3)


def kernel(x, emb, W1, b1, W2, b2, W3, b3):
    W1b = W1[K * W:]
    W1t = W1[:K * W].reshape(W, K, H).transpose(2, 1, 0).reshape(H, K * W)
    W1t = W1t.astype(jnp.bfloat16)
    xt = jnp.transpose(x, (2, 1, 0)).astype(jnp.bfloat16)
    xt = xt.reshape(N, W, 2, _HB).transpose(0, 1, 3, 2)
    xt_packed = lax.bitcast_convert_type(xt, jnp.int32)
    xt_packed = xt_packed.reshape(N, W * _HB)

    topk, projT = _topk_embproj(emb, W1b, b1)
    projT3 = projT.reshape(H, N // _NB, _NB).transpose(1, 0, 2)
    topk_flat = topk.reshape(-1)
    W2tb = W2.T.astype(jnp.bfloat16)
    W3t = W3.T
    blocks_per_s = _NCN // _NB
    outs = []
    for s in range(_S):
        g_s = _sc_gather(xt_packed,
                         topk_flat[s * _NCN * K:(s + 1) * _NCN * K])
        g_s = g_s.reshape(_NCN, K * W, _HB)
        pT_s = projT3[s * blocks_per_s:(s + 1) * blocks_per_s]
        outs.append(_mlp(g_s, pT_s, W1t, W2tb, b2, W3t, b3))
    out2 = jnp.concatenate(outs, axis=0)
    return out2.reshape(N, B).T
